# baseline (speedup 1.0000x reference)
import functools

import jax
import jax.numpy as jnp
from jax import lax
from jax.experimental import pallas as pl
from jax.experimental.pallas import tpu as pltpu
from jax.experimental.pallas import tpu_sc as plsc

N_ROWS = 16384
D = 64
P = 3
NC = 2
NS = 16
NW = NC * NS
CHUNK = 104
CHUNKS_PER_W = 5
B_PER_W = CHUNK * CHUNKS_PER_W
B_TOTAL = B_PER_W * NW

@functools.cache
def _sc_gather_kernel():
    mesh = plsc.VectorSubcoreMesh(core_axis_name="c", subcore_axis_name="s")

    @functools.partial(
        pl.kernel,
        mesh=mesh,
        out_type=jax.ShapeDtypeStruct((B_TOTAL, D), jnp.float32),
        scratch_types=[
            pltpu.VMEM((B_PER_W,), jnp.int32),
            pltpu.VMEM((B_PER_W, D), jnp.float32),
            pltpu.SemaphoreType.DMA,
        ],
        compiler_params=pltpu.CompilerParams(use_tc_tiling_on_sc=False),
    )
    def _sc_gather(table_hbm, idx_hbm, out_hbm, idx_v, rows_v, sem):
        wid = lax.axis_index("s") * NC + lax.axis_index("c")
        pltpu.sync_copy(idx_hbm.at[pl.ds(wid * B_PER_W, B_PER_W)], idx_v)
        copies = []
        for j in range(CHUNKS_PER_W):
            copies.append(
                pltpu.async_copy(
                    table_hbm.at[idx_v.at[pl.ds(j * CHUNK, CHUNK)]],
                    rows_v.at[pl.ds(j * CHUNK, CHUNK)],
                    sem,
                )
            )
        for c in copies:
            c.wait()
        pltpu.sync_copy(rows_v, out_hbm.at[pl.ds(wid * B_PER_W, B_PER_W)])

    return _sc_gather


BLK = 2048
_GRID = N_ROWS // BLK


def _tc_body(attach_ref, te_ref, fcw_ref, fcb_ref, fc2t_ref, fc2b_ref, out_ref):
    hi = lax.Precision.HIGHEST
    tr = fcb_ref[...]
    for p in range(P):
        tr = tr + jnp.dot(te_ref[p:p + 1, :], fcw_ref[p],
                          preferred_element_type=jnp.float32, precision=hi)
    c = jnp.dot(tr, fc2t_ref[...],
                preferred_element_type=jnp.float32, precision=hi) + fc2b_ref[...]
    out_ref[...] = c - jnp.dot(attach_ref[...], fc2t_ref[...],
                               preferred_element_type=jnp.float32, precision=hi)


_tc_call = pl.pallas_call(
    _tc_body,
    grid=(_GRID,),
    in_specs=[
        pl.BlockSpec((BLK, D), lambda i: (i, 0)),
        pl.BlockSpec((P, D), lambda i: (0, 0)),
        pl.BlockSpec((P, D, D), lambda i: (0, 0, 0)),
        pl.BlockSpec((1, D), lambda i: (0, 0)),
        pl.BlockSpec((D, P + 1), lambda i: (0, 0)),
        pl.BlockSpec((1, P + 1), lambda i: (0, 0)),
    ],
    out_specs=pl.BlockSpec((BLK, P + 1), lambda i: (i, 0)),
    out_shape=jax.ShapeDtypeStruct((N_ROWS, P + 1), jnp.float32),
)


def kernel(term, attach_terms, device, table, fc_W, fc_b, fc2_W, fc2_b):
    del device
    idx = jnp.concatenate([
        attach_terms.astype(jnp.int32),
        term.astype(jnp.int32),
        jnp.zeros((B_TOTAL - N_ROWS - P,), jnp.int32),
    ])
    g = _sc_gather_kernel()(table, idx)
    attach_embed = g[:N_ROWS]
    term_embed = g[N_ROWS:N_ROWS + P]
    fc_Wr = fc_W.reshape(D, P, D).transpose(1, 2, 0)
    return _tc_call(attach_embed, term_embed, fc_Wr,
                    fc_b.reshape(1, D), fc2_W.T, fc2_b.reshape(1, P + 1))
